# Initial kernel scaffold; baseline (speedup 1.0000x reference)
import jax
import jax.numpy as jnp
from jax.experimental import pallas as pl


def kernel(h0, coord0, g0, params, edge_index, batch):
    raise NotImplementedError("write your pallas kernel here")


# speedup vs baseline: 3.6236x; 3.6236x over previous
import functools

import jax
import jax.numpy as jnp
from jax import lax
from jax.experimental import pallas as pl
from jax.experimental.pallas import tpu as pltpu
from jax.experimental.pallas import tpu_sc as plsc

N = 10000
E = 160000
D = 128
B = 64
NC = 2
NS = 16
NW = NC * NS
CG = 128
NCH = E // CG
CS = 128
CPC = (E // NC) // CS
RA = 624
_RCOPIES = ((0, 128), (128, 128), (256, 128), (384, 128), (512, 112))
_TAIL = NS * RA
_TSZ = N - _TAIL
BE = 2000
BN_ = 2000
GE = E // BE
GN = N // BN_
EPS = 1e-5

f32 = jnp.float32
i32 = jnp.int32


def _sds(shape, dtype=f32):
    return jax.ShapeDtypeStruct(shape, dtype)


def _sig(x):
    return 1.0 / (1.0 + jnp.exp(-x))


def _silu(x):
    return x * _sig(x)


def _elu1(x):
    return jnp.where(x > 0, x, jnp.exp(x) - 1.0)


def _elu01(x):
    return jnp.where(x > 0, x, 0.1 * (jnp.exp(x) - 1.0))


def _split3(a, b, dims):
    bf16 = jnp.bfloat16
    ah = a.astype(bf16)
    al = (a - ah.astype(f32)).astype(bf16)
    bh = b.astype(bf16)
    bl = (b - bh.astype(f32)).astype(bf16)
    d = lambda x, y: lax.dot_general(x, y, dims, preferred_element_type=f32)
    return d(ah, bl) + d(al, bh) + d(ah, bh)


def _dot(a, b):
    return lax.dot_general(a, b, (((1,), (0,)), ((), ())),
                           preferred_element_type=f32)


def _dotx(a, b):
    return _split3(a, b, (((1,), (0,)), ((), ())))


def _dott(a, b):
    return _split3(a, b, (((0,), (0,)), ((), ())))


def _vmesh():
    return plsc.VectorSubcoreMesh(core_axis_name="c", subcore_axis_name="s")


def _sc_gather_edge(h, xs, ys, zs, row, col):
    @functools.partial(
        pl.kernel,
        out_type=(_sds((E, D)), _sds((E, D)), _sds((E, 4))),
        mesh=_vmesh(),
        compiler_params=pltpu.CompilerParams(needs_layout_passes=False),
        scratch_types=[
            pltpu.VMEM((CG,), i32), pltpu.VMEM((CG,), i32),
            pltpu.VMEM((CG, D), f32), pltpu.VMEM((CG, D), f32),
            pltpu.VMEM((CG, 4), f32),
            pltpu.VMEM((N,), f32), pltpu.VMEM((N,), f32),
            pltpu.VMEM((N,), f32),
            pltpu.SemaphoreType.DMA,
        ],
    )
    def k(h_h, xs_h, ys_h, zs_h, row_h, col_h, ohr, ohc, oc4,
          idr, idc, bhr, bhc, b4, xv, yv, zv, sem):
        c = lax.axis_index("c")
        s = lax.axis_index("s")
        w = s * NC + c
        pltpu.sync_copy(xs_h, xv)
        pltpu.sync_copy(ys_h, yv)
        pltpu.sync_copy(zs_h, zv)
        nk = jnp.where(w < NCH % NW, NCH // NW + 1, NCH // NW)

        def body(kk, carry):
            off = (w + kk * NW) * CG
            pltpu.sync_copy(row_h.at[pl.ds(off, CG)], idr)
            pltpu.sync_copy(col_h.at[pl.ds(off, CG)], idc)
            pltpu.async_copy(h_h.at[idr], bhr, sem).wait()
            pltpu.async_copy(h_h.at[idc], bhc, sem).wait()
            for g in range(CG // 16):
                rv = idr[pl.ds(g * 16, 16)]
                cv = idc[pl.ds(g * 16, 16)]
                dx = plsc.load_gather(xv, [rv]) - plsc.load_gather(xv, [cv])
                dy = plsc.load_gather(yv, [rv]) - plsc.load_gather(yv, [cv])
                dz = plsc.load_gather(zv, [rv]) - plsc.load_gather(zv, [cv])
                rad = dx * dx + dy * dy + dz * dz
                ev = jax.lax.iota(i32, 16) + g * 16
                for comp, val in ((0, dx), (1, dy), (2, dz), (3, rad)):
                    plsc.store_scatter(
                        b4, [ev, jnp.full((16,), comp, i32)], val)
            pltpu.sync_copy(bhr, ohr.at[pl.ds(off, CG)])
            pltpu.sync_copy(bhc, ohc.at[pl.ds(off, CG)])
            pltpu.sync_copy(b4, oc4.at[pl.ds(off, CG)])
            return carry

        lax.fori_loop(0, nk, body, 0)

    return k(h, xs, ys, zs, row, col)


def _zero_vmem(zb, rows, lanes):
    def z(i, carry):
        r = i // (lanes // 16)
        cc = i % (lanes // 16)
        zb[r, pl.ds(cc * 16, 16)] = jnp.zeros((16,), f32)
        return carry
    lax.fori_loop(0, rows * (lanes // 16), z, 0)


def _zero_acc(zb, acc, s):
    for off_, sz in _RCOPIES:
        pltpu.sync_copy(zb.at[pl.ds(0, sz)], acc.at[pl.ds(s * RA + off_, sz)])

    @pl.when(s == 0)
    def _():
        pltpu.sync_copy(zb.at[pl.ds(0, _TSZ)], acc.at[pl.ds(_TAIL, _TSZ)])


def _acc_out(acc, out, s, base=0):
    for off_, sz in _RCOPIES:
        pltpu.sync_copy(acc.at[pl.ds(s * RA + off_, sz)],
                        out.at[pl.ds(base + s * RA + off_, sz)])

    @pl.when(s == 0)
    def _():
        pltpu.sync_copy(acc.at[pl.ds(_TAIL, _TSZ)],
                        out.at[pl.ds(base + _TAIL, _TSZ)])


def _sc_scatter_edge(m, aux, row):
    @functools.partial(
        pl.kernel,
        out_type=(_sds((N, D)), _sds((N, D))),
        mesh=_vmesh(),
        scratch_types=[
            pltpu.VMEM((CS,), i32),
            pltpu.VMEM((CS, D), f32),
            pltpu.VMEM((128, D), f32),
            pltpu.VMEM_SHARED((N, D), f32),
            pltpu.SemaphoreType.DMA,
        ],
    )
    def k(m_h, aux_h, row_h, om, oa, idx, bm, zb, accm, sem):
        c = lax.axis_index("c")
        s = lax.axis_index("s")
        _zero_vmem(zb, 128, D)
        _zero_acc(zb, accm, s)
        plsc.subcore_barrier()

        nk = jnp.where(s < NCH % NS, NCH // NS + 1, NCH // NS)

        def loop(src_h):
            def body(kk, carry):
                off = (s + kk * NS) * CS
                pltpu.sync_copy(row_h.at[pl.ds(off, CS)], idx)
                pltpu.sync_copy(src_h.at[pl.ds(off, CS)], bm)
                pltpu.sync_copy(bm, accm.at[idx], add=True)
                return carry
            lax.fori_loop(0, nk, body, 0)

        @pl.when(c == 0)
        def _():
            loop(m_h)

        @pl.when(c == 1)
        def _():
            loop(aux_h)

        plsc.subcore_barrier()

        @pl.when(c == 0)
        def _():
            _acc_out(accm, om, s)

        @pl.when(c == 1)
        def _():
            _acc_out(accm, oa, s)

    return k(m, aux, row)


def _sc_gin_gather_scatter(h, row, col):
    @functools.partial(
        pl.kernel,
        out_type=_sds((NC * N, D)),
        mesh=_vmesh(),
        scratch_types=[
            pltpu.VMEM((CS,), i32), pltpu.VMEM((CS,), i32),
            pltpu.VMEM((CS, D), f32),
            pltpu.VMEM((128, D), f32),
            pltpu.VMEM_SHARED((N, D), f32),
            pltpu.SemaphoreType.DMA,
        ],
    )
    def k(h_h, row_h, col_h, om, idr, idc, buf, zb, accm, sem):
        c = lax.axis_index("c")
        s = lax.axis_index("s")
        _zero_vmem(zb, 128, D)
        _zero_acc(zb, accm, s)
        plsc.subcore_barrier()

        nk = jnp.where(s < CPC % NS, CPC // NS + 1, CPC // NS)

        def body(kk, carry):
            off = (c * CPC + s + kk * NS) * CS
            pltpu.sync_copy(row_h.at[pl.ds(off, CS)], idr)
            pltpu.sync_copy(col_h.at[pl.ds(off, CS)], idc)
            pltpu.async_copy(h_h.at[idr], buf, sem).wait()
            pltpu.sync_copy(buf, accm.at[idc], add=True)
            return carry
        lax.fori_loop(0, nk, body, 0)
        plsc.subcore_barrier()
        _acc_out(accm, om, s, base=c * N)

    return k(h, row, col)


def _full(shape):
    return pl.BlockSpec(shape, lambda i: (0,) * len(shape))


def _seg_max_block(ids, x, acc):
    g_lo = ids[0, 0]
    g_hi = ids[BN_ - 1, 0]
    rowg = lax.broadcasted_iota(i32, (B, 1), 0)

    def gbody(g, a):
        sel = jnp.where(ids == g, x, -jnp.inf)
        mrow = jnp.max(sel, axis=0, keepdims=True)
        return jnp.maximum(a, jnp.where(rowg == g, mrow, -jnp.inf))

    return lax.fori_loop(g_lo, g_hi + 1, gbody, acc)


def _k_edge(hr, hc, cd4, w1h, w1c, w1r, b1, w2, b2, cw1, cb1, cw2):
    def body(hr_r, hc_r, c4_r, w1h_r, w1c_r, w1r_r, b1_r, w2_r, b2_r,
             cw1_r, cb1_r, cw2_r, om_r, oa_r):
        c4 = c4_r[...]
        radial = c4[:, 3:4]
        t1 = (_dot(hr_r[...], w1h_r[...]) + _dot(hc_r[...], w1c_r[...])
              + radial * w1r_r[...] + b1_r[...])
        m = _silu(_dot(_silu(t1), w2_r[...]) + b2_r[...])
        u = _silu(_dot(m, cw1_r[...]) + cb1_r[...])
        cm = _dot(u, cw2_r[...])
        om_r[...] = m
        cdw = jnp.concatenate([c4, jnp.zeros((BE, D - 4), f32)], axis=1)
        lane = lax.broadcasted_iota(i32, (BE, D), 1)
        oa_r[...] = jnp.where(lane == 3, 1.0, cdw * cm)

    return pl.pallas_call(
        body,
        grid=(GE,),
        in_specs=[
            pl.BlockSpec((BE, D), lambda i: (i, 0)),
            pl.BlockSpec((BE, D), lambda i: (i, 0)),
            pl.BlockSpec((BE, 4), lambda i: (i, 0)),
            _full((D, D)), _full((D, D)), _full((1, D)), _full((1, D)),
            _full((D, D)), _full((1, D)),
            _full((D, D)), _full((1, D)), _full((D, 1)),
        ],
        out_specs=[
            pl.BlockSpec((BE, D), lambda i: (i, 0)),
            pl.BlockSpec((BE, D), lambda i: (i, 0)),
        ],
        out_shape=(_sds((E, D)), _sds((E, D))),
    )(hr, hc, cd4, w1h, w1c, w1r, b1, w2, b2, cw1, cb1, cw2)


def _k_node(h, cpad, pm, pa, w1h, w1a, b1, w2, b2):
    def body(h_r, pm_r, pa_r, cp_r, w1h_r, w1a_r, b1_r, w2_r, b2_r,
             oh_r, oc_r, st_r):
        agg = pm_r[...]
        auxs = pa_r[...]
        cnt = jnp.maximum(auxs[:, 3:4], 1.0)
        lane = lax.broadcasted_iota(i32, (BN_, 16), 1)
        oc_r[...] = cp_r[...] + jnp.where(lane < 3, auxs[:, :16], 0.0) / cnt
        o = _silu(_dot(h_r[...], w1h_r[...]) + _dot(agg, w1a_r[...])
                  + b1_r[...])
        hpre = _elu1(h_r[...] + _dot(o, w2_r[...]) + b2_r[...])
        oh_r[...] = hpre

        @pl.when(pl.program_id(0) == 0)
        def _():
            st_r[...] = jnp.zeros((8, D), f32)
        st_r[0:1, :] += jnp.sum(hpre, axis=0, keepdims=True)
        st_r[1:2, :] += jnp.sum(hpre * hpre, axis=0, keepdims=True)

    return pl.pallas_call(
        body,
        grid=(GN,),
        in_specs=[
            pl.BlockSpec((BN_, D), lambda i: (i, 0)),
            pl.BlockSpec((BN_, D), lambda i: (i, 0)),
            pl.BlockSpec((BN_, D), lambda i: (i, 0)),
            pl.BlockSpec((BN_, 16), lambda i: (i, 0)),
            _full((D, D)), _full((D, D)), _full((1, D)),
            _full((D, D)), _full((1, D)),
        ],
        out_specs=[
            pl.BlockSpec((BN_, D), lambda i: (i, 0)),
            pl.BlockSpec((BN_, 16), lambda i: (i, 0)),
            _full((8, D)),
        ],
        out_shape=(_sds((N, D)), _sds((N, 16)), _sds((8, D))),
    )(h, pm, pa, cpad, w1h, w1a, b1, w2, b2)


def _k_var(x, stats):
    def body(x_r, st_r, ov_r):
        mu = st_r[0:1, :] / N
        d = x_r[...] - mu

        @pl.when(pl.program_id(0) == 0)
        def _():
            ov_r[...] = jnp.zeros((8, D), f32)
        ov_r[0:1, :] += jnp.sum(d * d, axis=0, keepdims=True)

    return pl.pallas_call(
        body,
        grid=(GN,),
        in_specs=[pl.BlockSpec((BN_, D), lambda i: (i, 0)), _full((8, D))],
        out_specs=_full((8, D)),
        out_shape=_sds((8, D)),
    )(x, stats)


def _k_bnm(hpre, stats, vstats, bng, bnb, mw, mb, batch3):
    def body(hp_r, st_r, vs_r, bng_r, bnb_r, mw_r, mb_r, ids_r,
             ob_r, omx_r, oms_r, omc_r):
        mu = st_r[0:1, :] / N
        var = vs_r[0:1, :] / N
        hbn = bng_r[...] * (hp_r[...] - mu) / jnp.sqrt(var + EPS) + bnb_r[...]
        ob_r[...] = hbn
        xm = _elu1(_dot(hbn, mw_r[...]) + mb_r[...])

        ids = ids_r[0]
        gids = lax.broadcasted_iota(i32, (BN_, B), 1)
        onehot = (ids == gids).astype(f32)

        @pl.when(pl.program_id(0) == 0)
        def _():
            omx_r[...] = jnp.full((B, D), -jnp.inf, f32)
            oms_r[...] = jnp.zeros((B, D), f32)
            omc_r[...] = jnp.zeros((B, 1), f32)
        oms_r[...] += _dott(onehot, xm)
        omc_r[...] += _dott(onehot, jnp.ones((BN_, 1), f32))
        omx_r[...] = _seg_max_block(ids, xm, omx_r[...])

    return pl.pallas_call(
        body,
        grid=(GN,),
        in_specs=[
            pl.BlockSpec((BN_, D), lambda i: (i, 0)),
            _full((8, D)), _full((8, D)), _full((1, D)), _full((1, D)),
            _full((D, D)), _full((1, D)),
            pl.BlockSpec((1, BN_, 1), lambda i: (i, 0, 0)),
        ],
        out_specs=[
            pl.BlockSpec((BN_, D), lambda i: (i, 0)),
            _full((B, D)), _full((B, D)), _full((B, 1)),
        ],
        out_shape=(_sds((N, D)), _sds((B, D)), _sds((B, D)), _sds((B, 1))),
    )(hpre, stats, vstats, bng, bnb, mw, mb, batch3)


def _k_pert(hbn, mx, ms, mc, cvw, cvb, batch3):
    def body(hb_r, mx_r, ms_r, mc_r, cvw_r, cvb_r, ids_r, oh_r):
        cnt = jnp.maximum(mc_r[...], 1.0)
        pert = _elu1(cvw_r[0, 0] * mx_r[...] + cvw_r[0, 1] * (ms_r[...] / cnt)
                     + cvb_r[0, 0])
        pert = jnp.where(mc_r[...] > 0, pert, 0.0)
        ids = ids_r[0]
        gids = lax.broadcasted_iota(i32, (BN_, B), 1)
        onehot = (ids == gids).astype(f32)
        oh_r[...] = hb_r[...] + _dotx(onehot, pert)

    return pl.pallas_call(
        body,
        grid=(GN,),
        in_specs=[
            pl.BlockSpec((BN_, D), lambda i: (i, 0)),
            _full((B, D)), _full((B, D)), _full((B, 1)),
            _full((1, 2)), _full((1, 1)),
            pl.BlockSpec((1, BN_, 1), lambda i: (i, 0, 0)),
        ],
        out_specs=pl.BlockSpec((BN_, D), lambda i: (i, 0)),
        out_shape=_sds((N, D)),
    )(hbn, mx, ms, mc, cvw, cvb, batch3)


def _k_gin_a(h, pg, w1, b1):
    def body(h_r, p0_r, p1_r, w1_r, b1_r, ot_r, st_r):
        t = _dot(h_r[...] + p0_r[...] + p1_r[...], w1_r[...]) + b1_r[...]
        ot_r[...] = t

        @pl.when(pl.program_id(0) == 0)
        def _():
            st_r[...] = jnp.zeros((8, D), f32)
        st_r[0:1, :] += jnp.sum(t, axis=0, keepdims=True)
        st_r[1:2, :] += jnp.sum(t * t, axis=0, keepdims=True)

    return pl.pallas_call(
        body,
        grid=(GN,),
        in_specs=[
            pl.BlockSpec((BN_, D), lambda i: (i, 0)),
            pl.BlockSpec((BN_, D), lambda i: (i, 0)),
            pl.BlockSpec((BN_, D), lambda i: (i + GN, 0)),
            _full((D, D)), _full((1, D)),
        ],
        out_specs=[pl.BlockSpec((BN_, D), lambda i: (i, 0)), _full((8, D))],
        out_shape=(_sds((N, D)), _sds((8, D))),
    )(h, pg, pg, w1, b1)


def _k_gin_b(t, stats, vstats, bng, bnb, w2, b2):
    def body(t_r, st_r, vs_r, bng_r, bnb_r, w2_r, b2_r, oh_r):
        mu = st_r[0:1, :] / N
        var = vs_r[0:1, :] / N
        y = bng_r[...] * (t_r[...] - mu) / jnp.sqrt(var + EPS) + bnb_r[...]
        y = jnp.maximum(y, 0.0)
        oh_r[...] = _elu01(_dot(y, w2_r[...]) + b2_r[...])

    return pl.pallas_call(
        body,
        grid=(GN,),
        in_specs=[
            pl.BlockSpec((BN_, D), lambda i: (i, 0)),
            _full((8, D)), _full((8, D)), _full((1, D)), _full((1, D)),
            _full((D, D)), _full((1, D)),
        ],
        out_specs=pl.BlockSpec((BN_, D), lambda i: (i, 0)),
        out_shape=_sds((N, D)),
    )(t, stats, vstats, bng, bnb, w2, b2)


def _k_pool(h, batch3):
    def body(h_r, ids_r, omx_r, oms_r, omc_r):
        x = h_r[...]
        ids = ids_r[0]
        gids = lax.broadcasted_iota(i32, (BN_, B), 1)
        onehot = (ids == gids).astype(f32)

        @pl.when(pl.program_id(0) == 0)
        def _():
            omx_r[...] = jnp.full((B, D), -jnp.inf, f32)
            oms_r[...] = jnp.zeros((B, D), f32)
            omc_r[...] = jnp.zeros((B, 1), f32)
        oms_r[...] += _dott(onehot, x)
        omc_r[...] += _dott(onehot, jnp.ones((BN_, 1), f32))
        omx_r[...] = _seg_max_block(ids, x, omx_r[...])

    return pl.pallas_call(
        body,
        grid=(GN,),
        in_specs=[
            pl.BlockSpec((BN_, D), lambda i: (i, 0)),
            pl.BlockSpec((1, BN_, 1), lambda i: (i, 0, 0)),
        ],
        out_specs=[_full((B, D)), _full((B, D)), _full((B, 1))],
        out_shape=(_sds((B, D)), _sds((B, D)), _sds((B, 1))),
    )(h, batch3)


def _k_cls(hmx, hms, hmc, g0, wmean, wmax, wg, b1, bng, bnb, w2, b2):
    def body(mx_r, ms_r, mc_r, g0_r, wme_r, wmx_r, wg_r, b1_r,
             bng_r, bnb_r, w2_r, b2_r, op_r):
        cnt = jnp.maximum(mc_r[...], 1.0)
        mean = ms_r[...] / cnt
        x = (_dot(mean, wme_r[...]) + _dot(mx_r[...], wmx_r[...])
             + _dot(g0_r[...], wg_r[...]) + b1_r[...])
        x = _elu01(x)
        mu = jnp.mean(x, axis=0, keepdims=True)
        xc = x - mu
        var = jnp.mean(xc * xc, axis=0, keepdims=True)
        x = bng_r[...] * xc / jnp.sqrt(var + EPS) + bnb_r[...]
        z = _dot(x, w2_r[...]) + b2_r[...]
        z = z - jnp.max(z, axis=1, keepdims=True)
        ez = jnp.exp(z)
        op_r[...] = ez / jnp.sum(ez, axis=1, keepdims=True)

    return pl.pallas_call(
        body,
        grid=(1,),
        in_specs=[
            _full((B, D)), _full((B, D)), _full((B, 1)), _full((B, 16)),
            _full((D, D)), _full((D, D)), _full((16, D)), _full((1, D)),
            _full((1, D)), _full((1, D)), _full((D, 10)), _full((1, 10)),
        ],
        out_specs=_full((B, 10)),
        out_shape=_sds((B, 10)),
    )(hmx, hms, hmc, g0, wmean, wmax, wg, b1, bng, bnb, w2, b2)


def kernel(h0, coord0, g0, params, edge_index, batch):
    row = edge_index[0]
    col = edge_index[1]
    batch3 = batch.reshape(GN, BN_, 1)
    r1 = lambda v: v.reshape(1, -1)

    h = h0
    cpad = jnp.concatenate([coord0, jnp.zeros((N, 13), f32)], axis=1)

    for l in range(2):
        p = lambda s: params['eq%d_%s' % (l, s)]
        xs = cpad[:, 0]
        ys = cpad[:, 1]
        zs = cpad[:, 2]
        hr, hc, cd4 = _sc_gather_edge(h, xs, ys, zs, row, col)
        ew1 = p('e_w1')
        m, aux = _k_edge(hr, hc, cd4,
                         ew1[:D], ew1[D:2 * D], ew1[2 * D:2 * D + 1],
                         r1(p('e_b1')), p('e_w2'), r1(p('e_b2')),
                         p('c_w1'), r1(p('c_b1')), p('c_w2'))
        pm, pa = _sc_scatter_edge(m, aux, row)
        nw1 = p('n_w1')
        hpre, cpad, stats = _k_node(h, cpad, pm, pa,
                                    nw1[:D], nw1[D:], r1(p('n_b1')),
                                    p('n_w2'), r1(p('n_b2')))
        vstats = _k_var(hpre, stats)
        hbn, mx, ms, mc = _k_bnm(hpre, stats, vstats,
                                 r1(p('bn_g')), r1(p('bn_b')),
                                 p('m_w'), r1(p('m_b')), batch3)
        h = _k_pert(hbn, mx, ms, mc, p('cv_w').reshape(1, 2),
                    p('cv_b').reshape(1, 1), batch3)

    for l in range(2):
        p = lambda s: params['gin%d_%s' % (l, s)]
        pg = _sc_gin_gather_scatter(h, row, col)
        t, stats = _k_gin_a(h, pg, p('w1'), r1(p('b1')))
        vstats = _k_var(t, stats)
        h = _k_gin_b(t, stats, vstats, r1(p('bn_g')), r1(p('bn_b')),
                     p('w2'), r1(p('b2')))

    hmx, hms, hmc = _k_pool(h, batch3)
    cw1 = params['cls_w1']
    probs = _k_cls(hmx, hms, hmc, g0,
                   cw1[:D], cw1[D:2 * D], cw1[2 * D:],
                   r1(params['cls_b1']), r1(params['cls_bn_g']),
                   r1(params['cls_bn_b']), params['cls_w2'],
                   r1(params['cls_b2']))
    return probs
